# Initial kernel scaffold; baseline (speedup 1.0000x reference)
import jax
import jax.numpy as jnp
from jax.experimental import pallas as pl


def kernel(x, W):
    raise NotImplementedError("write your pallas kernel here")


# speedup vs baseline: 5.9164x; 5.9164x over previous
import functools

import jax
import jax.numpy as jnp
from jax import lax
from jax.experimental import pallas as pl
from jax.experimental.pallas import tpu as pltpu
from jax.experimental.pallas import tpu_sc as plsc

B, S, K = 4096, 200, 6
N = B * S
D = 128
NC, NS, L = 2, 16, 16
NW = NC * NS
ROWS_PER_W = N // NW
C = 128
CHUNKS = ROWS_PER_W // C

_mesh = plsc.VectorSubcoreMesh(core_axis_name="c", subcore_axis_name="s")


@functools.partial(
    pl.kernel,
    mesh=_mesh,
    out_type=jax.ShapeDtypeStruct((N, D), jnp.float32),
    scratch_types=[
        pltpu.VMEM((K, C), jnp.int32),
        pltpu.VMEM((K, C, D), jnp.float32),
        pltpu.SemaphoreType.DMA,
    ],
)
def _sc_lookup_sum(w_hbm, xt_hbm, out_hbm, idx_v, buf_v, gsem):
    wid = lax.axis_index("s") * NC + lax.axis_index("c")
    base0 = wid * ROWS_PER_W

    def chunk_body(g, carry):
        base = base0 + g * C
        pltpu.sync_copy(xt_hbm.at[:, pl.ds(base, C)], idx_v)
        copies = [
            pltpu.async_copy(w_hbm.at[idx_v.at[k]], buf_v.at[k], gsem)
            for k in range(K)
        ]
        for cp in copies:
            cp.wait()

        def row_body(r, rcarry):
            for c in range(D // L):
                sl = pl.ds(c * L, L)
                acc = buf_v[0, r, sl]
                for k in range(1, K):
                    acc = acc + buf_v[k, r, sl]
                buf_v[0, r, sl] = acc
            return rcarry

        lax.fori_loop(0, C, row_body, 0, unroll=2)
        pltpu.sync_copy(buf_v.at[0], out_hbm.at[pl.ds(base, C), :])
        return carry

    lax.fori_loop(0, CHUNKS, chunk_body, 0)


def kernel(x, W):
    xt = jnp.moveaxis(x.reshape(N, K).astype(jnp.int32), -1, 0)
    out = _sc_lookup_sum(W.astype(jnp.float32), xt)
    return out.reshape(B, S, D)
